# baseline (speedup 1.0000x reference)
import functools

import jax
import jax.numpy as jnp
from jax import lax
from jax.experimental import pallas as pl
from jax.experimental.pallas import tpu as pltpu
from jax.experimental.pallas import tpu_sc as plsc

_L = 16


def _sc_body(tpw, c, g, d, x_h, idx_h, mask_h, pe_h, out_h,
             idx_v, mask_v, idxp_v, xb, rows, sem):
    nc = 2
    wid = lax.axis_index("s") * nc + lax.axis_index("c")
    base = wid * tpw

    pltpu.sync_copy(idx_h.at[pl.ds(base, tpw)], idx_v)
    pltpu.sync_copy(mask_h.at[pl.ds(base, tpw)], mask_v)

    def remap(i, _):
        sl = pl.ds(i * _L, _L)
        idxp_v[sl] = jnp.where(mask_v[sl] != 0, idx_v[sl], g)
        return 0

    lax.fori_loop(0, tpw // _L, remap, 0)

    dsl = d // _L

    def step(s_i, _):
        tb = base + s_i * c
        pltpu.sync_copy(x_h.at[pl.ds(tb, c)], xb)
        pltpu.async_copy(pe_h.at[idxp_v.at[pl.ds(s_i * c, c)]], rows, sem).wait()

        def add(i, _):
            t = i // dsl
            j = i % dsl
            sl = pl.ds(j * _L, _L)
            xb[t, sl] = xb[t, sl] + rows[t, sl]
            return 0

        lax.fori_loop(0, c * dsl, add, 0)
        pltpu.sync_copy(xb, out_h.at[pl.ds(tb, c)])
        return 0

    lax.fori_loop(0, tpw // c, step, 0)


def kernel(x, local_indices, group_mask, pe):
    b, s, d = x.shape
    g = pe.shape[0]
    n = b * s
    nw = 32
    tpw = n // nw
    c = 16

    xf = x.reshape(n, d)
    idx = local_indices.reshape(n).astype(jnp.int32)
    mask = group_mask.reshape(n).astype(jnp.int32)
    pe_ext = jnp.concatenate([pe, jnp.zeros((8, d), pe.dtype)], axis=0)

    mesh = plsc.VectorSubcoreMesh(core_axis_name="c", subcore_axis_name="s")
    run = pl.kernel(
        functools.partial(_sc_body, tpw, c, g, d),
        out_type=jax.ShapeDtypeStruct((n, d), x.dtype),
        mesh=mesh,
        scratch_types=[
            pltpu.VMEM((tpw,), jnp.int32),
            pltpu.VMEM((tpw,), jnp.int32),
            pltpu.VMEM((tpw,), jnp.int32),
            pltpu.VMEM((c, d), jnp.float32),
            pltpu.VMEM((c, d), jnp.float32),
            pltpu.SemaphoreType.DMA,
        ],
    )
    out = run(xf, idx, mask, pe_ext)
    return out.reshape(b, s, d)
